# Initial kernel scaffold; baseline (speedup 1.0000x reference)
import jax
import jax.numpy as jnp
from jax.experimental import pallas as pl


def kernel(column):
    raise NotImplementedError("write your pallas kernel here")


# speedup vs baseline: 2.2345x; 2.2345x over previous
import jax
import jax.numpy as jnp
from jax.experimental import pallas as pl

PAD = 0.0
B, L = 16384, 200
BS = 512


def _trim_block(x_ref, dense_ref, len_ref):
    x = x_ref[...]
    dense_ref[:, :L] = x
    dense_ref[:, L:] = jnp.zeros((BS, 1), x.dtype)
    pos1 = jax.lax.broadcasted_iota(jnp.int32, (BS, L), 1) + 1
    lengths = jnp.max(jnp.where(x != PAD, pos1, 0), axis=1)
    len_ref[...] = lengths + 1


@jax.jit
def kernel(column):
    grid = (B // BS,)
    dense, row_lengths = pl.pallas_call(
        _trim_block,
        grid=grid,
        in_specs=[pl.BlockSpec((BS, L), lambda i: (i, 0))],
        out_specs=[
            pl.BlockSpec((BS, L + 1), lambda i: (i, 0)),
            pl.BlockSpec((BS,), lambda i: (i,)),
        ],
        out_shape=[
            jax.ShapeDtypeStruct((B, L + 1), column.dtype),
            jax.ShapeDtypeStruct((B,), jnp.int32),
        ],
    )(column)
    return dense, row_lengths


# speedup vs baseline: 2.5924x; 1.1602x over previous
import jax
import jax.numpy as jnp
from jax.experimental import pallas as pl

PAD = 0.0
B, L = 16384, 200
BS = 1024


def _trim_block(x_ref, dense_ref, len_ref):
    x = x_ref[...]
    dense_ref[:, :L] = x
    dense_ref[:, L:] = jnp.zeros((BS, 1), x.dtype)
    pos1 = jax.lax.broadcasted_iota(jnp.int32, (BS, L), 1) + 1
    lengths = jnp.max(jnp.where(x != PAD, pos1, 0), axis=1)
    len_ref[...] = lengths + 1


@jax.jit
def kernel(column):
    grid = (B // BS,)
    dense, row_lengths = pl.pallas_call(
        _trim_block,
        grid=grid,
        in_specs=[pl.BlockSpec((BS, L), lambda i: (i, 0))],
        out_specs=[
            pl.BlockSpec((BS, L + 1), lambda i: (i, 0)),
            pl.BlockSpec((BS,), lambda i: (i,)),
        ],
        out_shape=[
            jax.ShapeDtypeStruct((B, L + 1), column.dtype),
            jax.ShapeDtypeStruct((B,), jnp.int32),
        ],
    )(column)
    return dense, row_lengths


# speedup vs baseline: 2.8553x; 1.1014x over previous
import jax
import jax.numpy as jnp
from jax.experimental import pallas as pl

PAD = 0.0
B, L = 16384, 200
BS = 2048


def _trim_block(x_ref, dense_ref, len_ref):
    x = x_ref[...]
    dense_ref[:, :L] = x
    dense_ref[:, L:] = jnp.zeros((BS, 1), x.dtype)
    pos1 = jax.lax.broadcasted_iota(jnp.int32, (BS, L), 1) + 1
    lengths = jnp.max(jnp.where(x != PAD, pos1, 0), axis=1)
    len_ref[...] = lengths + 1


@jax.jit
def kernel(column):
    grid = (B // BS,)
    dense, row_lengths = pl.pallas_call(
        _trim_block,
        grid=grid,
        in_specs=[pl.BlockSpec((BS, L), lambda i: (i, 0))],
        out_specs=[
            pl.BlockSpec((BS, L + 1), lambda i: (i, 0)),
            pl.BlockSpec((BS,), lambda i: (i,)),
        ],
        out_shape=[
            jax.ShapeDtypeStruct((B, L + 1), column.dtype),
            jax.ShapeDtypeStruct((B,), jnp.int32),
        ],
    )(column)
    return dense, row_lengths


# speedup vs baseline: 2.9338x; 1.0275x over previous
import jax
import jax.numpy as jnp
from jax.experimental import pallas as pl

PAD = 0.0
B, L = 16384, 200
BS = 4096


def _trim_block(x_ref, dense_ref, len_ref):
    x = x_ref[...]
    dense_ref[:, :L] = x
    dense_ref[:, L:] = jnp.zeros((BS, 1), x.dtype)
    pos1 = jax.lax.broadcasted_iota(jnp.int32, (BS, L), 1) + 1
    lengths = jnp.max(jnp.where(x != PAD, pos1, 0), axis=1)
    len_ref[...] = lengths + 1


@jax.jit
def kernel(column):
    grid = (B // BS,)
    dense, row_lengths = pl.pallas_call(
        _trim_block,
        grid=grid,
        in_specs=[pl.BlockSpec((BS, L), lambda i: (i, 0))],
        out_specs=[
            pl.BlockSpec((BS, L + 1), lambda i: (i, 0)),
            pl.BlockSpec((BS,), lambda i: (i,)),
        ],
        out_shape=[
            jax.ShapeDtypeStruct((B, L + 1), column.dtype),
            jax.ShapeDtypeStruct((B,), jnp.int32),
        ],
    )(column)
    return dense, row_lengths
